# baseline (speedup 1.0000x reference)
import functools

import jax
import jax.numpy as jnp
from jax import lax
from jax.experimental import pallas as pl
from jax.experimental.pallas import tpu as pltpu
from jax.experimental.pallas import tpu_sc as plsc

BATCH = 4
SEQ = 2048
DM = 1024
ROWS = BATCH * SEQ
X_ELEMS = ROWS * DM
EMB_ELEMS = SEQ * DM

BS = 2048


def _tc_body(x_ref, emb_ref, o_ref):
    off = (pl.program_id(0) * BS) % SEQ
    o_ref[:, :] = x_ref[:, :] + emb_ref[pl.ds(off, BS), :]


def _tc_kernel(x, emb_table):
    xf = x.reshape(ROWS, DM)
    out = pl.pallas_call(
        _tc_body,
        grid=(ROWS // BS,),
        in_specs=[
            pl.BlockSpec((BS, DM), lambda i: (i, 0)),
            pl.BlockSpec((SEQ, DM), lambda i: (0, 0)),
        ],
        out_specs=pl.BlockSpec((BS, DM), lambda i: (i, 0)),
        out_shape=jax.ShapeDtypeStruct((ROWS, DM), x.dtype),
    )(xf, emb_table)
    return out.reshape(BATCH, SEQ, DM)


_NC = 2
_NS = 16
_NW = _NC * _NS
_W_ELEMS = X_ELEMS // _NW
_CHUNK = 16 * DM
_NCHUNK = _W_ELEMS // _CHUNK


def _sc_body(x_hbm, emb_hbm, out_hbm, xv, ev, ov, xsem, esem, osem):
    wid = lax.axis_index("s") * _NC + lax.axis_index("c")
    base = wid * _W_ELEMS
    emb_base = lax.rem(base, EMB_ELEMS)

    def start_in(k):
        b = k % 2
        off = base + k * _CHUNK
        eoff = emb_base + k * _CHUNK
        xd = pltpu.async_copy(x_hbm.at[pl.ds(off, _CHUNK)], xv.at[b], xsem.at[b])
        ed = pltpu.async_copy(emb_hbm.at[pl.ds(eoff, _CHUNK)], ev.at[b], esem.at[b])
        return xd, ed

    descs = start_in(0)
    out_descs = [None, None]
    for k in range(_NCHUNK):
        b = k % 2
        nxt = None
        if k + 1 < _NCHUNK:
            if out_descs[(k + 1) % 2] is not None:
                out_descs[(k + 1) % 2].wait()
                out_descs[(k + 1) % 2] = None
            nxt = start_in(k + 1)
        xd, ed = descs
        xd.wait()
        ed.wait()

        @plsc.parallel_loop(0, _CHUNK, step=16, unroll=16)
        def _add(i):
            s = pl.ds(i, 16)
            ov[b, s] = xv[b, s] + ev[b, s]

        off = base + k * _CHUNK
        out_descs[b] = pltpu.async_copy(
            ov.at[b], out_hbm.at[pl.ds(off, _CHUNK)], osem.at[b])
        descs = nxt
    for d in out_descs:
        if d is not None:
            d.wait()


@functools.partial(jax.jit, static_argnums=())
def _sc_kernel(x, emb_table):
    mesh = plsc.VectorSubcoreMesh(core_axis_name="c", subcore_axis_name="s")
    run = pl.kernel(
        _sc_body,
        out_type=jax.ShapeDtypeStruct((X_ELEMS,), jnp.float32),
        mesh=mesh,
        scratch_types=[
            pltpu.VMEM((2, _CHUNK), jnp.float32),
            pltpu.VMEM((2, _CHUNK), jnp.float32),
            pltpu.VMEM((2, _CHUNK), jnp.float32),
            pltpu.SemaphoreType.DMA((2,)),
            pltpu.SemaphoreType.DMA((2,)),
            pltpu.SemaphoreType.DMA((2,)),
        ],
    )
    out = run(x.reshape(X_ELEMS), emb_table.reshape(EMB_ELEMS))
    return out.reshape(BATCH, SEQ, DM)


def kernel(x, emb_table):
    return _sc_kernel(x, emb_table)


# speedup vs baseline: 1.1105x; 1.1105x over previous
import functools

import jax
import jax.numpy as jnp
from jax import lax
from jax.experimental import pallas as pl
from jax.experimental.pallas import tpu as pltpu
from jax.experimental.pallas import tpu_sc as plsc

BATCH = 4
SEQ = 2048
DM = 1024
ROWS = BATCH * SEQ
X_ELEMS = ROWS * DM
EMB_ELEMS = SEQ * DM

BS = 2048


def _tc_body(x_ref, emb_ref, o_ref):
    off = (pl.program_id(0) * BS) % SEQ
    o_ref[:, :] = x_ref[:, :] + emb_ref[pl.ds(off, BS), :]


def _tc_kernel(x, emb_table):
    xf = x.reshape(ROWS, DM)
    out = pl.pallas_call(
        _tc_body,
        grid=(ROWS // BS,),
        in_specs=[
            pl.BlockSpec((BS, DM), lambda i: (i, 0)),
            pl.BlockSpec((SEQ, DM), lambda i: (0, 0)),
        ],
        out_specs=pl.BlockSpec((BS, DM), lambda i: (i, 0)),
        out_shape=jax.ShapeDtypeStruct((ROWS, DM), x.dtype),
    )(xf, emb_table)
    return out.reshape(BATCH, SEQ, DM)


_NC = 2
_NS = 16
_NW = _NC * _NS
_W_ELEMS = X_ELEMS // _NW
_CHUNK = 16 * DM
_NCHUNK = _W_ELEMS // _CHUNK
_NBUF = 3


def _sc_body(x_hbm, emb_hbm, out_hbm, xv0, xv1, xv2, ev0, ev1, ev2,
             xsem, esem, osem):
    xv = [xv0, xv1, xv2]
    ev = [ev0, ev1, ev2]
    wid = lax.axis_index("s") * _NC + lax.axis_index("c")
    base = wid * _W_ELEMS
    emb_base = lax.rem(base, EMB_ELEMS)

    def start_in(k):
        b = k % _NBUF
        off = base + k * _CHUNK
        eoff = emb_base + k * _CHUNK
        xd = pltpu.async_copy(x_hbm.at[pl.ds(off, _CHUNK)], xv[b], xsem.at[b])
        ed = pltpu.async_copy(emb_hbm.at[pl.ds(eoff, _CHUNK)], ev[b], esem.at[b])
        return xd, ed

    in_descs = [None] * _NBUF
    out_descs = [None] * _NBUF
    for k in range(_NBUF - 1):
        in_descs[k % _NBUF] = start_in(k)
    for k in range(_NCHUNK):
        b = k % _NBUF
        kn = k + _NBUF - 1
        if kn < _NCHUNK:
            bn = kn % _NBUF
            if out_descs[bn] is not None:
                out_descs[bn].wait()
                out_descs[bn] = None
            in_descs[bn] = start_in(kn)
        xd, ed = in_descs[b]
        xd.wait()
        ed.wait()
        in_descs[b] = None

        @plsc.parallel_loop(0, _CHUNK, step=16, unroll=16)
        def _add(i):
            s = pl.ds(i, 16)
            xv[b][s] = xv[b][s] + ev[b][s]

        off = base + k * _CHUNK
        if out_descs[b] is not None:
            out_descs[b].wait()
        out_descs[b] = pltpu.async_copy(
            xv[b], out_hbm.at[pl.ds(off, _CHUNK)], osem.at[b])
    for d in out_descs:
        if d is not None:
            d.wait()


@functools.partial(jax.jit, static_argnums=())
def _sc_kernel(x, emb_table):
    mesh = plsc.VectorSubcoreMesh(core_axis_name="c", subcore_axis_name="s")
    run = pl.kernel(
        _sc_body,
        out_type=jax.ShapeDtypeStruct((X_ELEMS,), jnp.float32),
        mesh=mesh,
        scratch_types=[
            pltpu.VMEM((_CHUNK,), jnp.float32),
            pltpu.VMEM((_CHUNK,), jnp.float32),
            pltpu.VMEM((_CHUNK,), jnp.float32),
            pltpu.VMEM((_CHUNK,), jnp.float32),
            pltpu.VMEM((_CHUNK,), jnp.float32),
            pltpu.VMEM((_CHUNK,), jnp.float32),
            pltpu.SemaphoreType.DMA((_NBUF,)),
            pltpu.SemaphoreType.DMA((_NBUF,)),
            pltpu.SemaphoreType.DMA((_NBUF,)),
        ],
    )
    out = run(x.reshape(X_ELEMS), emb_table.reshape(EMB_ELEMS))
    return out.reshape(BATCH, SEQ, DM)


def kernel(x, emb_table):
    return _sc_kernel(x, emb_table)


# speedup vs baseline: 5.7964x; 5.2195x over previous
import functools

import jax
import jax.numpy as jnp
from jax import lax
from jax.experimental import pallas as pl
from jax.experimental.pallas import tpu as pltpu
from jax.experimental.pallas import tpu_sc as plsc

BATCH = 4
SEQ = 2048
DM = 1024
ROWS = BATCH * SEQ
X_ELEMS = ROWS * DM
EMB_ELEMS = SEQ * DM

BS = 2048


def _tc_body(x_ref, emb_ref, o_ref):
    off = (pl.program_id(0) * BS) % SEQ
    o_ref[:, :] = x_ref[:, :] + emb_ref[pl.ds(off, BS), :]


def _tc_kernel(x, emb_table):
    xf = x.reshape(ROWS, DM)
    out = pl.pallas_call(
        _tc_body,
        grid=(ROWS // BS,),
        in_specs=[
            pl.BlockSpec((BS, DM), lambda i: (i, 0)),
            pl.BlockSpec((SEQ, DM), lambda i: (0, 0)),
        ],
        out_specs=pl.BlockSpec((BS, DM), lambda i: (i, 0)),
        out_shape=jax.ShapeDtypeStruct((ROWS, DM), x.dtype),
    )(xf, emb_table)
    return out.reshape(BATCH, SEQ, DM)


_NC = 2
_NS = 16
_NW = _NC * _NS
_W_ELEMS = X_ELEMS // _NW
_CHUNK = 16 * DM
_NCHUNK = _W_ELEMS // _CHUNK
_NBUF = 3


def _sc_body(x_hbm, emb_hbm, out_hbm, xv0, xv1, xv2, ev0, ev1, ev2,
             xsem, esem, osem):
    xv = [xv0, xv1, xv2]
    ev = [ev0, ev1, ev2]
    wid = lax.axis_index("s") * _NC + lax.axis_index("c")
    base = wid * _W_ELEMS
    emb_base = lax.rem(base, EMB_ELEMS)

    def start_in(k):
        b = k % _NBUF
        off = base + k * _CHUNK
        eoff = emb_base + k * _CHUNK
        xd = pltpu.async_copy(x_hbm.at[pl.ds(off, _CHUNK)], xv[b], xsem.at[b])
        ed = pltpu.async_copy(emb_hbm.at[pl.ds(eoff, _CHUNK)], ev[b], esem.at[b])
        return xd, ed

    in_descs = [None] * _NBUF
    out_descs = [None] * _NBUF
    for k in range(_NBUF - 1):
        in_descs[k % _NBUF] = start_in(k)
    for k in range(_NCHUNK):
        b = k % _NBUF
        kn = k + _NBUF - 1
        if kn < _NCHUNK:
            bn = kn % _NBUF
            if out_descs[bn] is not None:
                out_descs[bn].wait()
                out_descs[bn] = None
            in_descs[bn] = start_in(kn)
        xd, ed = in_descs[b]
        xd.wait()
        ed.wait()
        in_descs[b] = None

        @plsc.parallel_loop(0, _CHUNK, step=16, unroll=16)
        def _add(i):
            s = pl.ds(i, 16)
            xv[b][s] = xv[b][s] + ev[b][s]

        off = base + k * _CHUNK
        if out_descs[b] is not None:
            out_descs[b].wait()
        out_descs[b] = pltpu.async_copy(
            xv[b], out_hbm.at[pl.ds(off, _CHUNK)], osem.at[b])
    for d in out_descs:
        if d is not None:
            d.wait()


@functools.partial(jax.jit, static_argnums=())
def _sc_kernel(x, emb_table):
    mesh = plsc.VectorSubcoreMesh(core_axis_name="c", subcore_axis_name="s")
    run = pl.kernel(
        _sc_body,
        out_type=jax.ShapeDtypeStruct((X_ELEMS,), jnp.float32),
        mesh=mesh,
        scratch_types=[
            pltpu.VMEM((_CHUNK,), jnp.float32),
            pltpu.VMEM((_CHUNK,), jnp.float32),
            pltpu.VMEM((_CHUNK,), jnp.float32),
            pltpu.VMEM((_CHUNK,), jnp.float32),
            pltpu.VMEM((_CHUNK,), jnp.float32),
            pltpu.VMEM((_CHUNK,), jnp.float32),
            pltpu.SemaphoreType.DMA((_NBUF,)),
            pltpu.SemaphoreType.DMA((_NBUF,)),
            pltpu.SemaphoreType.DMA((_NBUF,)),
        ],
    )
    out = run(x.reshape(X_ELEMS), emb_table.reshape(EMB_ELEMS))
    return out.reshape(BATCH, SEQ, DM)


HBS = 1024


def _tc_body2(x_ref, emb_ref, o_ref):
    o_ref[0, :, :] = x_ref[0, :, :] + emb_ref[:, :]


def _tc_kernel2(x, emb_table):
    return pl.pallas_call(
        _tc_body2,
        grid=(SEQ // HBS, BATCH),
        in_specs=[
            pl.BlockSpec((1, HBS, DM), lambda i, j: (j, i, 0)),
            pl.BlockSpec((HBS, DM), lambda i, j: (i, 0)),
        ],
        out_specs=pl.BlockSpec((1, HBS, DM), lambda i, j: (j, i, 0)),
        out_shape=jax.ShapeDtypeStruct((BATCH, SEQ, DM), x.dtype),
    )(x, emb_table)


def kernel(x, emb_table):
    return _tc_kernel2(x, emb_table)


# speedup vs baseline: 6.2708x; 1.0818x over previous
import functools

import jax
import jax.numpy as jnp
from jax import lax
from jax.experimental import pallas as pl
from jax.experimental.pallas import tpu as pltpu
from jax.experimental.pallas import tpu_sc as plsc

BATCH = 4
SEQ = 2048
DM = 1024
ROWS = BATCH * SEQ
X_ELEMS = ROWS * DM
EMB_ELEMS = SEQ * DM

BS = 2048


def _tc_body(x_ref, emb_ref, o_ref):
    off = (pl.program_id(0) * BS) % SEQ
    o_ref[:, :] = x_ref[:, :] + emb_ref[pl.ds(off, BS), :]


def _tc_kernel(x, emb_table):
    xf = x.reshape(ROWS, DM)
    out = pl.pallas_call(
        _tc_body,
        grid=(ROWS // BS,),
        in_specs=[
            pl.BlockSpec((BS, DM), lambda i: (i, 0)),
            pl.BlockSpec((SEQ, DM), lambda i: (0, 0)),
        ],
        out_specs=pl.BlockSpec((BS, DM), lambda i: (i, 0)),
        out_shape=jax.ShapeDtypeStruct((ROWS, DM), x.dtype),
    )(xf, emb_table)
    return out.reshape(BATCH, SEQ, DM)


_NC = 2
_NS = 16
_NW = _NC * _NS
_W_ELEMS = X_ELEMS // _NW
_CHUNK = 16 * DM
_NCHUNK = _W_ELEMS // _CHUNK
_NBUF = 3


def _sc_body(x_hbm, emb_hbm, out_hbm, xv0, xv1, xv2, ev0, ev1, ev2,
             xsem, esem, osem):
    xv = [xv0, xv1, xv2]
    ev = [ev0, ev1, ev2]
    wid = lax.axis_index("s") * _NC + lax.axis_index("c")
    base = wid * _W_ELEMS
    emb_base = lax.rem(base, EMB_ELEMS)

    def start_in(k):
        b = k % _NBUF
        off = base + k * _CHUNK
        eoff = emb_base + k * _CHUNK
        xd = pltpu.async_copy(x_hbm.at[pl.ds(off, _CHUNK)], xv[b], xsem.at[b])
        ed = pltpu.async_copy(emb_hbm.at[pl.ds(eoff, _CHUNK)], ev[b], esem.at[b])
        return xd, ed

    in_descs = [None] * _NBUF
    out_descs = [None] * _NBUF
    for k in range(_NBUF - 1):
        in_descs[k % _NBUF] = start_in(k)
    for k in range(_NCHUNK):
        b = k % _NBUF
        kn = k + _NBUF - 1
        if kn < _NCHUNK:
            bn = kn % _NBUF
            if out_descs[bn] is not None:
                out_descs[bn].wait()
                out_descs[bn] = None
            in_descs[bn] = start_in(kn)
        xd, ed = in_descs[b]
        xd.wait()
        ed.wait()
        in_descs[b] = None

        @plsc.parallel_loop(0, _CHUNK, step=16, unroll=16)
        def _add(i):
            s = pl.ds(i, 16)
            xv[b][s] = xv[b][s] + ev[b][s]

        off = base + k * _CHUNK
        if out_descs[b] is not None:
            out_descs[b].wait()
        out_descs[b] = pltpu.async_copy(
            xv[b], out_hbm.at[pl.ds(off, _CHUNK)], osem.at[b])
    for d in out_descs:
        if d is not None:
            d.wait()


@functools.partial(jax.jit, static_argnums=())
def _sc_kernel(x, emb_table):
    mesh = plsc.VectorSubcoreMesh(core_axis_name="c", subcore_axis_name="s")
    run = pl.kernel(
        _sc_body,
        out_type=jax.ShapeDtypeStruct((X_ELEMS,), jnp.float32),
        mesh=mesh,
        scratch_types=[
            pltpu.VMEM((_CHUNK,), jnp.float32),
            pltpu.VMEM((_CHUNK,), jnp.float32),
            pltpu.VMEM((_CHUNK,), jnp.float32),
            pltpu.VMEM((_CHUNK,), jnp.float32),
            pltpu.VMEM((_CHUNK,), jnp.float32),
            pltpu.VMEM((_CHUNK,), jnp.float32),
            pltpu.SemaphoreType.DMA((_NBUF,)),
            pltpu.SemaphoreType.DMA((_NBUF,)),
            pltpu.SemaphoreType.DMA((_NBUF,)),
        ],
    )
    out = run(x.reshape(X_ELEMS), emb_table.reshape(EMB_ELEMS))
    return out.reshape(BATCH, SEQ, DM)


def kernel(x, emb_table):
    return _tc_kernel(x, emb_table)
